# baseline (speedup 1.0000x reference)
import functools

import jax
import jax.numpy as jnp
from jax import lax
from jax.experimental import pallas as pl
from jax.experimental.pallas import tpu as pltpu
from jax.experimental.pallas import tpu_sc as plsc

N_USERS = 100000
N_ITEMS = 1000
BATCH = 16384

NC = 2
NS = 16
L = 16
NW = NC * NS
BPW = BATCH // NW

NBKT = 8
CH_ROWS = 32
MAXCH = BPW // CH_ROWS + NBKT - 1
LISTCAP = MAXCH * CH_ROWS + CH_ROWS

_mesh = plsc.VectorSubcoreMesh(core_axis_name="c", subcore_axis_name="s")


@functools.partial(
    pl.kernel,
    out_type=jax.ShapeDtypeStruct((BATCH,), jnp.float32),
    mesh=_mesh,
    compiler_params=pltpu.CompilerParams(needs_layout_passes=False),
    scratch_types=[
        pltpu.VMEM((BPW,), jnp.int32),
        pltpu.VMEM((BPW,), jnp.int32),
        pltpu.VMEM((LISTCAP,), jnp.int32),
        pltpu.VMEM((LISTCAP,), jnp.int32),
        pltpu.VMEM((MAXCH * CH_ROWS, 128), jnp.float32),
        pltpu.VMEM((BPW,), jnp.float32),
        pltpu.SMEM((NBKT + 1,), jnp.int32),
        pltpu.SemaphoreType.DMA,
    ],
)
def _gather_kernel(prop_hbm, user_hbm, item_hbm, out_hbm,
                   user_v, item_v, ulist_v, jlist_v, rows_v, out_v,
                   cnt_s, sem):
    wid = lax.axis_index("s") * NC + lax.axis_index("c")
    base = wid * BPW

    pltpu.sync_copy(user_hbm.at[pl.ds(base, BPW)], user_v)
    pltpu.sync_copy(item_hbm.at[pl.ds(base, BPW)], item_v)

    zeros = jnp.zeros((L,), jnp.int32)
    for i in range(0, LISTCAP, L):
        ulist_v[pl.ds(i, L)] = zeros
    for c in range(NBKT):
        cnt_s[c] = 0
    cnt_s[NBKT] = (NBKT - 1) * 128

    iota = lax.broadcasted_iota(jnp.int32, (L,), 0)

    for g in range(BPW // L):
        ct = item_v[pl.ds(g * L, L)] >> 7
        for c in range(NBKT):
            pc = plsc.all_reduce_population_count(ct == c)
            cnt_s[c] = cnt_s[c] + pc[0]

    counts = [cnt_s[c] for c in range(NBKT)]
    bases = [jnp.int32(0)]
    for c in range(NBKT):
        bases.append(bases[c] + ((counts[c] + (CH_ROWS - 1))
                                 & ~(CH_ROWS - 1)))
    for c in range(NBKT):
        cnt_s[c] = 0

    for g in range(BPW // L):
        u = user_v[pl.ds(g * L, L)]
        ct = item_v[pl.ds(g * L, L)] >> 7
        jv = iota + (g * L)
        for c in range(NBKT):
            mask = ct == c
            pos = bases[c] + cnt_s[c] + plsc.cumsum(
                mask.astype(jnp.int32)) - 1
            pos = jnp.where(mask, pos, 0)
            plsc.store_scatter(ulist_v, [pos], u, mask=mask)
            plsc.store_scatter(jlist_v, [pos], jv, mask=mask)
            pc = plsc.all_reduce_population_count(mask)
            cnt_s[c] = cnt_s[c] + pc[0]

    t = jnp.int32(0)
    starts = []
    for c in range(NBKT):
        nch = (counts[c] + (CH_ROWS - 1)) >> 5
        starts.append(t)
        if c < NBKT - 1:
            colstart = c * 128
        else:
            colstart = pl.multiple_of(cnt_s[NBKT], 128)

        def issue(k, tt, c=c, colstart=colstart):
            lstart = pl.multiple_of(bases[c] + k * CH_ROWS, CH_ROWS)
            dstart = pl.multiple_of(tt * CH_ROWS, CH_ROWS)
            pltpu.async_copy(
                prop_hbm.at[ulist_v.at[pl.ds(lstart, CH_ROWS)],
                            pl.ds(colstart, 128)],
                rows_v.at[pl.ds(dstart, CH_ROWS), :],
                sem,
            )
            return tt + 1

        t = lax.fori_loop(0, nch, issue, t)

    def drain(_, carry):
        pltpu.make_async_copy(
            prop_hbm.at[pl.ds(0, CH_ROWS), pl.ds(0, 128)],
            rows_v.at[pl.ds(0, CH_ROWS), :],
            sem,
        ).wait()
        return carry

    lax.fori_loop(0, t, drain, jnp.int32(0))

    for c in range(NBKT):
        n_c = counts[c]
        cs = starts[c]

        def select(k, carry, c=c, cs=cs, n_c=n_c):
            pvec = iota + k * L
            mask = pvec < n_c
            pv = jnp.where(mask, pvec, 0)
            jv = plsc.load_gather(jlist_v, [pv + bases[c]])
            jv = jnp.where(mask, jv, 0)
            itv = plsc.load_gather(item_v, [jv])
            col = jnp.where(mask, itv - c * 128, 0)
            slot = jnp.where(mask,
                             (cs + (pv >> 5)) * CH_ROWS + (pv & (CH_ROWS - 1)),
                             0)
            val = plsc.load_gather(rows_v, [slot, col])
            plsc.store_scatter(out_v, [jv], val, mask=mask)
            return carry

        lax.fori_loop(0, (n_c + (L - 1)) >> 4, select, jnp.int32(0))

    pltpu.sync_copy(out_v, out_hbm.at[pl.ds(base, BPW)])


def kernel(prop, user_idx, item_idx):
    return _gather_kernel(prop, user_idx, item_idx)


# speedup vs baseline: 13.9678x; 13.9678x over previous
import functools

import jax
import jax.numpy as jnp
from jax import lax
from jax.experimental import pallas as pl
from jax.experimental.pallas import tpu as pltpu
from jax.experimental.pallas import tpu_sc as plsc

N_USERS = 100000
N_ITEMS = 1000
BATCH = 16384

NC = 2
NS = 16
L = 16
NW = NC * NS
BPW = BATCH // NW

_mesh = plsc.VectorSubcoreMesh(core_axis_name="c", subcore_axis_name="s")


@functools.partial(
    pl.kernel,
    out_type=jax.ShapeDtypeStruct((BATCH,), jnp.float32),
    mesh=_mesh,
    compiler_params=pltpu.CompilerParams(needs_layout_passes=False),
    scratch_types=[
        pltpu.VMEM((BPW,), jnp.int32),
        pltpu.VMEM((BPW,), jnp.int32),
        pltpu.VMEM((8 * BPW,), jnp.int32),
        pltpu.VMEM((BPW, 128), jnp.float32),
        pltpu.VMEM((BPW,), jnp.float32),
        pltpu.SemaphoreType.DMA,
    ],
)
def _gather_kernel(propT_hbm, user_hbm, item_hbm, out_hbm,
                   user_v, item_v, il8_v, rows_v, out_v, sem):
    wid = lax.axis_index("s") * NC + lax.axis_index("c")
    base = wid * BPW

    pltpu.sync_copy(user_hbm.at[pl.ds(base, BPW)], user_v)
    pltpu.sync_copy(item_hbm.at[pl.ds(base, BPW)], item_v)

    iota = lax.broadcasted_iota(jnp.int32, (L,), 0)
    for g in range(BPW // L):
        itv = item_v[pl.ds(g * L, L)]
        plsc.store_scatter(il8_v, [(iota + g * L) * 8], itv)

    def issue(g, carry):
        gbase = pl.multiple_of(g * L, L)
        uv = user_v[pl.ds(gbase, L)]
        wv = (uv >> 7) * 128
        for lane in range(L):
            p = g * L + lane
            lstart = pl.multiple_of(p * 8, 8)
            w = pl.multiple_of(wv[lane], 128)
            pltpu.async_copy(
                propT_hbm.at[il8_v.at[pl.ds(lstart, 1)], pl.ds(w, 128)],
                rows_v.at[pl.ds(p, 1), :],
                sem,
            )
        return carry

    lax.fori_loop(0, BPW // L, issue, jnp.int32(0))

    def drain(_, carry):
        pltpu.make_async_copy(
            propT_hbm.at[pl.ds(0, 1), pl.ds(0, 128)],
            rows_v.at[pl.ds(0, 1), :],
            sem,
        ).wait()
        return carry

    lax.fori_loop(0, BPW, drain, jnp.int32(0))

    for g in range(BPW // L):
        jv = iota + g * L
        col = user_v[pl.ds(g * L, L)] & 127
        out_v[pl.ds(g * L, L)] = plsc.load_gather(rows_v, [jv, col])

    pltpu.sync_copy(out_v, out_hbm.at[pl.ds(base, BPW)])


def kernel(prop, user_idx, item_idx):
    return _gather_kernel(jnp.swapaxes(prop, 0, 1), user_idx, item_idx)


# speedup vs baseline: 15.1577x; 1.0852x over previous
import functools

import jax
import jax.numpy as jnp
from jax import lax
from jax.experimental import pallas as pl
from jax.experimental.pallas import tpu as pltpu
from jax.experimental.pallas import tpu_sc as plsc

N_USERS = 100000
N_ITEMS = 1000
BATCH = 16384

NC = 2
NS = 16
L = 16
NW = NC * NS
BPW = BATCH // NW

_mesh = plsc.VectorSubcoreMesh(core_axis_name="c", subcore_axis_name="s")


@functools.partial(
    pl.kernel,
    out_type=jax.ShapeDtypeStruct((BATCH,), jnp.float32),
    mesh=_mesh,
    compiler_params=pltpu.CompilerParams(needs_layout_passes=False),
    scratch_types=[
        pltpu.VMEM((BPW,), jnp.int32),
        pltpu.VMEM((BPW,), jnp.int32),
        pltpu.VMEM((8 * BPW,), jnp.int32),
        pltpu.VMEM((BPW, 128), jnp.float32),
        pltpu.VMEM((BPW,), jnp.float32),
        pltpu.SemaphoreType.DMA,
    ],
)
def _gather_kernel(propT_hbm, user_hbm, item_hbm, out_hbm,
                   user_v, item_v, il8_v, rows_v, out_v, sem):
    wid = lax.axis_index("s") * NC + lax.axis_index("c")
    base = wid * BPW

    pltpu.sync_copy(user_hbm.at[pl.ds(base, BPW)], user_v)
    pltpu.sync_copy(item_hbm.at[pl.ds(base, BPW)], item_v)

    iota = lax.broadcasted_iota(jnp.int32, (L,), 0)
    for g in range(BPW // L):
        itv = item_v[pl.ds(g * L, L)]
        plsc.store_scatter(il8_v, [(iota + g * L) * 8], itv)

    def issue(g, carry):
        gbase = pl.multiple_of(g * L, L)
        uv = user_v[pl.ds(gbase, L)]
        wv = (uv >> 7) * 128
        for lane in range(L):
            p = g * L + lane
            lstart = pl.multiple_of(p * 8, 8)
            w = pl.multiple_of(wv[lane], 128)
            pltpu.async_copy(
                propT_hbm.at[il8_v.at[pl.ds(lstart, 1)], pl.ds(w, 128)],
                rows_v.at[pl.ds(p, 1), :],
                sem,
            )
        return carry

    lax.fori_loop(0, BPW // L, issue, jnp.int32(0))

    pltpu.make_async_copy(
        propT_hbm.at[pl.ds(0, BPW), pl.ds(0, 128)],
        rows_v,
        sem,
    ).wait()

    for g in range(BPW // L):
        jv = iota + g * L
        col = user_v[pl.ds(g * L, L)] & 127
        out_v[pl.ds(g * L, L)] = plsc.load_gather(rows_v, [jv, col])

    pltpu.sync_copy(out_v, out_hbm.at[pl.ds(base, BPW)])


def kernel(prop, user_idx, item_idx):
    return _gather_kernel(jnp.swapaxes(prop, 0, 1), user_idx, item_idx)
